# Initial kernel scaffold; baseline (speedup 1.0000x reference)
import jax
import jax.numpy as jnp
from jax.experimental import pallas as pl


def kernel(x, W, b, gamma, beta):
    raise NotImplementedError("write your pallas kernel here")


# speedup vs baseline: 10.2209x; 10.2209x over previous
import functools

import jax
import jax.numpy as jnp
from jax import lax
from jax.experimental import pallas as pl
from jax.experimental.pallas import tpu as pltpu
from jax.experimental.pallas import tpu_sc as plsc

_K = 20
_NC, _NS, _L = 2, 16, 16
_NW = _NC * _NS


def _topk_body(R, N, K, x_full_ref, x_tile_ref, idx_ref):
    b = pl.program_id(0)
    xt = x_full_ref[0]
    xc = x_tile_ref[0]
    g = lax.dot_general(xc, xt, (((0,), (0,)), ((), ())),
                        preferred_element_type=jnp.float32)
    xxf = jnp.sum(xt * xt, axis=0, keepdims=True)
    xxc = jnp.sum(xc * xc, axis=0)[:, None]
    dist = 2.0 * g - xxc - xxf
    iota = lax.broadcasted_iota(jnp.int32, (R, N), 1)
    cols = []
    for t in range(K):
        m = jnp.max(dist, axis=1, keepdims=True)
        ji = jnp.min(jnp.where(dist == m, iota, N), axis=1, keepdims=True)
        cols.append(ji)
        if t < K - 1:
            dist = jnp.where(iota == ji, -jnp.inf, dist)
    idx_ref[0] = jnp.concatenate(cols, axis=1) + b * N


def _run_topk(x, R, K):
    B, C, N = x.shape
    return pl.pallas_call(
        functools.partial(_topk_body, R, N, K),
        grid=(B, N // R),
        in_specs=[
            pl.BlockSpec((1, C, N), lambda b, r: (b, 0, 0)),
            pl.BlockSpec((1, C, R), lambda b, r: (b, 0, r)),
        ],
        out_specs=pl.BlockSpec((1, R, K), lambda b, r: (b, r, 0)),
        out_shape=jax.ShapeDtypeStruct((B, N, K), jnp.int32),
    )(x, x)


def _pq_body(x_ref, w_ref, bias_ref, pT_ref, qT_ref):
    xb = x_ref[0]
    w = w_ref[...]
    c = xb.shape[0]
    w1 = w[:, :c]
    w2 = w[:, c:]
    dn = (((0,), (1,)), ((), ()))
    qT = lax.dot_general(xb, w2, dn, preferred_element_type=jnp.float32)
    n, o = qT.shape
    qT_ref[0] = jnp.concatenate(
        [qT, jnp.zeros((n, 128 - o), jnp.float32)], axis=1)
    pT = lax.dot_general(xb, w1 - w2, dn, preferred_element_type=jnp.float32)
    pT_ref[0] = pT + bias_ref[...][None, :]


def _run_pq(x, w, bias):
    B, C, N = x.shape
    O = w.shape[0]
    return pl.pallas_call(
        _pq_body,
        grid=(B,),
        in_specs=[
            pl.BlockSpec((1, C, N), lambda b: (b, 0, 0)),
            pl.BlockSpec(w.shape, lambda b: (0, 0)),
            pl.BlockSpec(bias.shape, lambda b: (0,)),
        ],
        out_specs=[
            pl.BlockSpec((1, N, O), lambda b: (b, 0, 0)),
            pl.BlockSpec((1, N, 128), lambda b: (b, 0, 0)),
        ],
        out_shape=[jax.ShapeDtypeStruct((B, N, O), jnp.float32),
                   jax.ShapeDtypeStruct((B, N, 128), jnp.float32)],
    )(x, w, bias)


def _sc_gather_body(K, G, RPW, O, qT_hbm, idx_hbm,
                    s_hbm, s2_hbm, mx_hbm, mn_hbm,
                    idx_v, gath_v, s_v, s2_v, mx_v, mn_v, sem):
    wid = lax.axis_index("s") * _NC + lax.axis_index("c")
    base = wid * RPW
    n_idx_rows = (G * K) // 128
    pltpu.sync_copy(idx_hbm.at[pl.ds(wid * ((RPW * K) // 128),
                                     (RPW * K) // 128)], idx_v)

    def chunk(ci, carry):
        r0 = base + ci * G
        copies = [
            pltpu.async_copy(qT_hbm.at[idx_v.at[ci * n_idx_rows + i]],
                             gath_v.at[pl.ds(i * 128, 128)], sem)
            for i in range(n_idx_rows)
        ]
        for cp in copies:
            cp.wait()

        def row(g, carry2):
            for c4 in range(O // _L):
                sl = pl.ds(c4 * _L, _L)
                v = gath_v[g * K, sl]
                s, s2, mx, mn = v, v * v, v, v
                for j in range(1, K):
                    v = gath_v[g * K + j, sl]
                    s = s + v
                    s2 = s2 + v * v
                    mx = jnp.maximum(mx, v)
                    mn = jnp.minimum(mn, v)
                s_v[g, sl] = s
                s2_v[g, sl] = s2
                mx_v[g, sl] = mx
                mn_v[g, sl] = mn
            return carry2

        lax.fori_loop(0, G, row, 0)
        pltpu.sync_copy(s_v, s_hbm.at[pl.ds(r0, G)])
        pltpu.sync_copy(s2_v, s2_hbm.at[pl.ds(r0, G)])
        pltpu.sync_copy(mx_v, mx_hbm.at[pl.ds(r0, G)])
        pltpu.sync_copy(mn_v, mn_hbm.at[pl.ds(r0, G)])
        return carry

    lax.fori_loop(0, RPW // G, chunk, 0)


def _run_sc_gather(qT2, idx2d, K, O):
    BN = qT2.shape[0]
    G = 32
    RPW = BN // _NW
    mesh = plsc.VectorSubcoreMesh(core_axis_name="c", subcore_axis_name="s",
                                  num_cores=_NC, num_subcores=_NS)
    out = jax.ShapeDtypeStruct((BN, O), jnp.float32)
    kern = pl.kernel(
        functools.partial(_sc_gather_body, K, G, RPW, O),
        out_type=[out, out, out, out],
        mesh=mesh,
        scratch_types=[
            pltpu.VMEM(((RPW * K) // 128, 128), jnp.int32),
            pltpu.VMEM((G * K, 128), jnp.float32),
            pltpu.VMEM((G, O), jnp.float32),
            pltpu.VMEM((G, O), jnp.float32),
            pltpu.VMEM((G, O), jnp.float32),
            pltpu.VMEM((G, O), jnp.float32),
            pltpu.SemaphoreType.DMA,
        ],
    )
    return kern(qT2, idx2d)


def _stats_body(K, pT_ref, s_ref, s2_ref, sx_ref, sx2_ref):
    p = pT_ref[...]
    s = s_ref[...]
    s2 = s2_ref[...]
    kf = float(K)
    sx_ref[...] = jnp.sum(kf * p + s, axis=0, keepdims=True)
    sx2_ref[...] = jnp.sum(kf * p * p + 2.0 * p * s + s2, axis=0,
                           keepdims=True)


def _run_stats(pT2, s, s2, K):
    BN, O = pT2.shape
    out = jax.ShapeDtypeStruct((1, O), jnp.float32)
    return pl.pallas_call(
        functools.partial(_stats_body, K),
        out_shape=[out, out],
    )(pT2, s, s2)


def _final_body(pT_ref, mx_ref, mn_ref, scale_ref, shift_ref, gpos_ref,
                out_ref):
    z = pT_ref[...] + jnp.where(gpos_ref[...] > 0.5, mx_ref[...], mn_ref[...])
    out_ref[...] = jnp.maximum(z * scale_ref[...] + shift_ref[...], 0.0)


def _run_final(pT2, mx, mn, scale, shift, gpos):
    BN, O = pT2.shape
    RE = min(2048, BN)
    row_spec = pl.BlockSpec((RE, O), lambda r: (r, 0))
    vec_spec = pl.BlockSpec((1, O), lambda r: (0, 0))
    return pl.pallas_call(
        _final_body,
        grid=(BN // RE,),
        in_specs=[row_spec, row_spec, row_spec, vec_spec, vec_spec, vec_spec],
        out_specs=row_spec,
        out_shape=jax.ShapeDtypeStruct((BN, O), jnp.float32),
    )(pT2, mx, mn, scale, shift, gpos)


def kernel(x, W, b, gamma, beta):
    B, C, N = x.shape
    O = W.shape[0]
    K = _K
    BN = B * N

    idx = _run_topk(x, 256, K)
    idx2d = idx.reshape((BN * K) // 128, 128)

    pT, qT = _run_pq(x, W, b)
    pT2 = pT.reshape(BN, O)
    qT2 = qT.reshape(BN, 128)

    s, s2, mx, mn = _run_sc_gather(qT2, idx2d, K, O)

    sx, sx2 = _run_stats(pT2, s, s2, K)
    cnt = float(BN * K)
    mean = sx[0] / cnt
    var = sx2[0] / cnt - mean * mean
    scale = gamma / jnp.sqrt(var + 1e-5)
    shift = beta - mean * scale
    gpos = (gamma >= 0).astype(jnp.float32)

    outE = _run_final(pT2, mx, mn, scale.reshape(1, O), shift.reshape(1, O),
                      gpos.reshape(1, O))
    return outE.reshape(B, N, O).transpose(0, 2, 1)


# speedup vs baseline: 11.6314x; 1.1380x over previous
import functools

import jax
import jax.numpy as jnp
from jax import lax
from jax.experimental import pallas as pl
from jax.experimental.pallas import tpu as pltpu
from jax.experimental.pallas import tpu_sc as plsc

_K = 20
_NC, _NS, _L = 2, 16, 16
_NW = _NC * _NS


def _topk_body(R, N, K, x_full_ref, x_tile_ref, idx_ref):
    b = pl.program_id(0)
    xt = x_full_ref[0]
    xc = x_tile_ref[0]
    g = lax.dot_general(xc, xt, (((0,), (0,)), ((), ())),
                        preferred_element_type=jnp.float32)
    xxf = jnp.sum(xt * xt, axis=0, keepdims=True)
    xxc = jnp.sum(xc * xc, axis=0)[:, None]
    dist = 2.0 * g - xxc - xxf
    iota = lax.broadcasted_iota(jnp.int32, (R, N), 1)
    cols = []
    for t in range(K):
        ji = jnp.argmax(dist, axis=1)[:, None].astype(jnp.int32)
        cols.append(ji)
        if t < K - 1:
            dist = jnp.where(iota == ji, -jnp.inf, dist)
    idx_ref[0] = jnp.concatenate(cols, axis=1) + b * N


def _run_topk(x, R, K):
    B, C, N = x.shape
    return pl.pallas_call(
        functools.partial(_topk_body, R, N, K),
        grid=(B, N // R),
        in_specs=[
            pl.BlockSpec((1, C, N), lambda b, r: (b, 0, 0)),
            pl.BlockSpec((1, C, R), lambda b, r: (b, 0, r)),
        ],
        out_specs=pl.BlockSpec((1, R, K), lambda b, r: (b, r, 0)),
        out_shape=jax.ShapeDtypeStruct((B, N, K), jnp.int32),
    )(x, x)


def _pq_body(x_ref, w_ref, bias_ref, pT_ref, qT_ref):
    xb = x_ref[0]
    w = w_ref[...]
    c = xb.shape[0]
    w1 = w[:, :c]
    w2 = w[:, c:]
    dn = (((0,), (1,)), ((), ()))
    qT = lax.dot_general(xb, w2, dn, preferred_element_type=jnp.float32)
    n, o = qT.shape
    qT_ref[0] = jnp.concatenate(
        [qT, jnp.zeros((n, 128 - o), jnp.float32)], axis=1)
    pT = lax.dot_general(xb, w1 - w2, dn, preferred_element_type=jnp.float32)
    pT_ref[0] = pT + bias_ref[...][None, :]


def _run_pq(x, w, bias):
    B, C, N = x.shape
    O = w.shape[0]
    return pl.pallas_call(
        _pq_body,
        grid=(B,),
        in_specs=[
            pl.BlockSpec((1, C, N), lambda b: (b, 0, 0)),
            pl.BlockSpec(w.shape, lambda b: (0, 0)),
            pl.BlockSpec(bias.shape, lambda b: (0,)),
        ],
        out_specs=[
            pl.BlockSpec((1, N, O), lambda b: (b, 0, 0)),
            pl.BlockSpec((1, N, 128), lambda b: (b, 0, 0)),
        ],
        out_shape=[jax.ShapeDtypeStruct((B, N, O), jnp.float32),
                   jax.ShapeDtypeStruct((B, N, 128), jnp.float32)],
    )(x, w, bias)


def _sc_gather_body(K, G, RPW, O, qT_hbm, idx_hbm,
                    s_hbm, s2_hbm, mx_hbm, mn_hbm,
                    idx_v, gath_v, s_v, s2_v, mx_v, mn_v, sem):
    wid = lax.axis_index("s") * _NC + lax.axis_index("c")
    base = wid * RPW
    n_idx_rows = (G * K) // 128
    pltpu.sync_copy(idx_hbm.at[pl.ds(wid * ((RPW * K) // 128),
                                     (RPW * K) // 128)], idx_v)

    def chunk(ci, carry):
        r0 = base + ci * G
        copies = [
            pltpu.async_copy(qT_hbm.at[idx_v.at[ci * n_idx_rows + i]],
                             gath_v.at[pl.ds(i * 128, 128)], sem)
            for i in range(n_idx_rows)
        ]
        for cp in copies:
            cp.wait()

        def row(g, carry2):
            for c4 in range(O // _L):
                sl = pl.ds(c4 * _L, _L)
                v = gath_v[g * K, sl]
                s, s2, mx, mn = v, v * v, v, v
                for j in range(1, K):
                    v = gath_v[g * K + j, sl]
                    s = s + v
                    s2 = s2 + v * v
                    mx = jnp.maximum(mx, v)
                    mn = jnp.minimum(mn, v)
                s_v[g, sl] = s
                s2_v[g, sl] = s2
                mx_v[g, sl] = mx
                mn_v[g, sl] = mn
            return carry2

        lax.fori_loop(0, G, row, 0)
        pltpu.sync_copy(s_v, s_hbm.at[pl.ds(r0, G)])
        pltpu.sync_copy(s2_v, s2_hbm.at[pl.ds(r0, G)])
        pltpu.sync_copy(mx_v, mx_hbm.at[pl.ds(r0, G)])
        pltpu.sync_copy(mn_v, mn_hbm.at[pl.ds(r0, G)])
        return carry

    lax.fori_loop(0, RPW // G, chunk, 0)


def _run_sc_gather(qT2, idx2d, K, O):
    BN = qT2.shape[0]
    G = 32
    RPW = BN // _NW
    mesh = plsc.VectorSubcoreMesh(core_axis_name="c", subcore_axis_name="s",
                                  num_cores=_NC, num_subcores=_NS)
    out = jax.ShapeDtypeStruct((BN, O), jnp.float32)
    kern = pl.kernel(
        functools.partial(_sc_gather_body, K, G, RPW, O),
        out_type=[out, out, out, out],
        mesh=mesh,
        scratch_types=[
            pltpu.VMEM(((RPW * K) // 128, 128), jnp.int32),
            pltpu.VMEM((G * K, 128), jnp.float32),
            pltpu.VMEM((G, O), jnp.float32),
            pltpu.VMEM((G, O), jnp.float32),
            pltpu.VMEM((G, O), jnp.float32),
            pltpu.VMEM((G, O), jnp.float32),
            pltpu.SemaphoreType.DMA,
        ],
    )
    return kern(qT2, idx2d)


def _stats_body(K, pT_ref, s_ref, s2_ref, sx_ref, sx2_ref):
    p = pT_ref[...]
    s = s_ref[...]
    s2 = s2_ref[...]
    kf = float(K)
    sx_ref[...] = jnp.sum(kf * p + s, axis=0, keepdims=True)
    sx2_ref[...] = jnp.sum(kf * p * p + 2.0 * p * s + s2, axis=0,
                           keepdims=True)


def _run_stats(pT2, s, s2, K):
    BN, O = pT2.shape
    out = jax.ShapeDtypeStruct((1, O), jnp.float32)
    return pl.pallas_call(
        functools.partial(_stats_body, K),
        out_shape=[out, out],
    )(pT2, s, s2)


def _final_body(pT_ref, mx_ref, mn_ref, scale_ref, shift_ref, gpos_ref,
                out_ref):
    z = pT_ref[...] + jnp.where(gpos_ref[...] > 0.5, mx_ref[...], mn_ref[...])
    out_ref[...] = jnp.maximum(z * scale_ref[...] + shift_ref[...], 0.0)


def _run_final(pT2, mx, mn, scale, shift, gpos):
    BN, O = pT2.shape
    RE = min(2048, BN)
    row_spec = pl.BlockSpec((RE, O), lambda r: (r, 0))
    vec_spec = pl.BlockSpec((1, O), lambda r: (0, 0))
    return pl.pallas_call(
        _final_body,
        grid=(BN // RE,),
        in_specs=[row_spec, row_spec, row_spec, vec_spec, vec_spec, vec_spec],
        out_specs=row_spec,
        out_shape=jax.ShapeDtypeStruct((BN, O), jnp.float32),
    )(pT2, mx, mn, scale, shift, gpos)


def kernel(x, W, b, gamma, beta):
    B, C, N = x.shape
    O = W.shape[0]
    K = _K
    BN = B * N

    idx = _run_topk(x, 256, K)
    idx2d = idx.reshape((BN * K) // 128, 128)

    pT, qT = _run_pq(x, W, b)
    pT2 = pT.reshape(BN, O)
    qT2 = qT.reshape(BN, 128)

    s, s2, mx, mn = _run_sc_gather(qT2, idx2d, K, O)

    sx, sx2 = _run_stats(pT2, s, s2, K)
    cnt = float(BN * K)
    mean = sx[0] / cnt
    var = sx2[0] / cnt - mean * mean
    scale = gamma / jnp.sqrt(var + 1e-5)
    shift = beta - mean * scale
    gpos = (gamma >= 0).astype(jnp.float32)

    outE = _run_final(pT2, mx, mn, scale.reshape(1, O), shift.reshape(1, O),
                      gpos.reshape(1, O))
    return outE.reshape(B, N, O).transpose(0, 2, 1)
